# baseline (speedup 1.0000x reference)
import functools

import jax
import jax.numpy as jnp
from jax import lax
from jax.experimental import pallas as pl
from jax.experimental.pallas import tpu as pltpu
from jax.experimental.pallas import tpu_sc as plsc

_NUM_CORES = 2
_NUM_SUBCORES = 16
_LANES = 16
_NW = _NUM_CORES * _NUM_SUBCORES


@functools.lru_cache(maxsize=None)
def _make_sc_kernel(batch: int, n_sites: int):
    b_per_w = batch // _NW
    assert batch % (8 * _NW) == 0
    mesh = plsc.VectorSubcoreMesh(
        core_axis_name="c", subcore_axis_name="s")

    @functools.partial(
        pl.kernel,
        out_type=(
            jax.ShapeDtypeStruct((batch,), jnp.float32),
            jax.ShapeDtypeStruct((batch,), jnp.float32),
        ),
        mesh=mesh,
        compiler_params=pltpu.CompilerParams(needs_layout_passes=False),
        scratch_types=[
            pltpu.VMEM((b_per_w * n_sites,), jnp.int32),
            pltpu.VMEM((b_per_w,), jnp.int32),
            pltpu.VMEM((b_per_w,), jnp.float32),
            pltpu.VMEM((b_per_w,), jnp.float32),
            pltpu.SemaphoreType.DMA,
        ],
    )
    def sc_kernel(x_hbm, real_hbm, imag_hbm, out_r, out_i,
                  xv, idxv, rv, iv, sem):
        wid = lax.axis_index("s") * _NUM_CORES + lax.axis_index("c")
        base = wid * b_per_w
        pltpu.sync_copy(
            x_hbm.at[pl.ds(base * n_sites, b_per_w * n_sites)], xv)

        lanes = lax.iota(jnp.int32, _LANES)

        def body(i, carry):
            rows = (i * _LANES + lanes) * n_sites
            acc = jnp.zeros((_LANES,), jnp.int32)
            for j in range(n_sites):
                xj = plsc.load_gather(xv, [rows + j])
                acc = acc * 2 + ((1 - xj) >> 1)
            off = pl.multiple_of(i * _LANES, _LANES)
            idxv[pl.ds(off, _LANES)] = acc
            return carry

        lax.fori_loop(0, b_per_w // _LANES, body, 0)

        pltpu.async_copy(real_hbm.at[idxv], rv, sem).wait()
        pltpu.async_copy(imag_hbm.at[idxv], iv, sem).wait()
        pltpu.sync_copy(rv, out_r.at[pl.ds(base, b_per_w)])
        pltpu.sync_copy(iv, out_i.at[pl.ds(base, b_per_w)])

    return sc_kernel


def kernel(x, real, imag):
    batch, n_sites = x.shape
    x_flat = x.reshape(batch * n_sites)
    r, i = _make_sc_kernel(batch, n_sites)(x_flat, real, imag)
    return lax.complex(r, i)


# speedup vs baseline: 1.1348x; 1.1348x over previous
import functools

import jax
import jax.numpy as jnp
from jax import lax
from jax.experimental import pallas as pl
from jax.experimental.pallas import tpu as pltpu
from jax.experimental.pallas import tpu_sc as plsc

_NUM_CORES = 2
_NUM_SUBCORES = 16
_LANES = 16
_NW = _NUM_CORES * _NUM_SUBCORES


@functools.lru_cache(maxsize=None)
def _make_sc_kernel(batch: int, n_sites: int):
    b_per_w = batch // _NW
    assert batch % (8 * _NW) == 0
    mesh = plsc.VectorSubcoreMesh(
        core_axis_name="c", subcore_axis_name="s")

    @functools.partial(
        pl.kernel,
        out_type=(
            jax.ShapeDtypeStruct((batch,), jnp.float32),
            jax.ShapeDtypeStruct((batch,), jnp.float32),
        ),
        mesh=mesh,
        compiler_params=pltpu.CompilerParams(needs_layout_passes=False),
        scratch_types=[
            pltpu.VMEM((b_per_w, n_sites), jnp.int32),
            pltpu.VMEM((b_per_w,), jnp.int32),
            pltpu.VMEM((b_per_w,), jnp.float32),
            pltpu.VMEM((b_per_w,), jnp.float32),
            pltpu.SemaphoreType.DMA,
        ],
    )
    def sc_kernel(x_hbm, real_hbm, imag_hbm, out_r, out_i,
                  xv, idxv, rv, iv, sem):
        wid = lax.axis_index("s") * _NUM_CORES + lax.axis_index("c")
        base = wid * b_per_w
        pltpu.sync_copy(x_hbm.at[pl.ds(base, b_per_w), :], xv)

        lanes = lax.iota(jnp.int32, _LANES)

        def body(i, carry):
            rows = i * _LANES + lanes
            acc = jnp.zeros((_LANES,), jnp.int32)
            for j in range(n_sites):
                cols = jnp.full((_LANES,), j, jnp.int32)
                xj = plsc.load_gather(xv, [rows, cols])
                acc = acc * 2 + ((1 - xj) >> 1)
            off = pl.multiple_of(i * _LANES, _LANES)
            idxv[pl.ds(off, _LANES)] = acc
            return carry

        lax.fori_loop(0, b_per_w // _LANES, body, 0)

        pltpu.async_copy(real_hbm.at[idxv], rv, sem).wait()
        pltpu.async_copy(imag_hbm.at[idxv], iv, sem).wait()
        pltpu.sync_copy(rv, out_r.at[pl.ds(base, b_per_w)])
        pltpu.sync_copy(iv, out_i.at[pl.ds(base, b_per_w)])

    return sc_kernel


def kernel(x, real, imag):
    batch, n_sites = x.shape
    r, i = _make_sc_kernel(batch, n_sites)(x, real, imag)
    return lax.complex(r, i)
